# baseline (speedup 1.0000x reference)
import jax
import jax.numpy as jnp
from jax.experimental import pallas as pl
from jax.experimental.pallas import tpu as pltpu

N = 262144
C_IN = 6
C = 256
NUM_CLS = 20

TILE = 4096

_f32 = jnp.float32


def _dot(a, b):
    return jax.lax.dot_general(a, b, (((1,), (0,)), ((), ())),
                               preferred_element_type=_f32)


def _stats_kernel(x_ref, wbb_ref, bbb_ref, w1_ref, b1_ref, out_ref):
    i = pl.program_id(0)
    x = x_ref[...]
    feat = _dot(x, wbb_ref[...]) + bbb_ref[0:1, :]
    h = _dot(feat, w1_ref[...]) + b1_ref[0:1, :]

    @pl.when(i == 0)
    def _():
        out_ref[...] = jnp.zeros_like(out_ref)

    s1 = jnp.sum(h, axis=0, keepdims=True)
    s2 = jnp.sum(h * h, axis=0, keepdims=True)
    out_ref[0:1, :] += s1
    out_ref[1:2, :] += s2


def _loss_kernel(x_ref, coord_ref, ic_ref, seg_ref, inst_ref,
                 wbb_ref, bbb_ref, w1_ref, b1_ref, bn_ref,
                 w2_ref, b2_ref, ws_ref, bs_ref, out_ref, acc_ref):
    i = pl.program_id(0)
    nsteps = pl.num_programs(0)

    @pl.when(i == 0)
    def _():
        for k in range(6):
            acc_ref[k] = 0.0

    x = x_ref[...]
    feat = _dot(x, wbb_ref[...]) + bbb_ref[0:1, :]
    h = _dot(feat, w1_ref[...]) + b1_ref[0:1, :]
    mu = bn_ref[0:1, :]
    sd = bn_ref[1:2, :]
    gam = bn_ref[2:3, :]
    bet = bn_ref[3:4, :]
    hn = jnp.maximum((h - mu) / sd * gam + bet, 0.0)
    bp8 = _dot(hn, w2_ref[...]) + b2_ref[0:1, :]
    bp3 = bp8[:, 0:3]

    bg3 = ic_ref[...] - coord_ref[...]
    inst = inst_ref[...]
    mask = (inst != -1).astype(_f32)

    bias_dist = jnp.sum(jnp.abs(bp3 - bg3), axis=1)
    acc_ref[2] += jnp.sum(bias_dist * mask)
    acc_ref[3] += jnp.sum(mask)

    bp_n = jnp.sqrt(jnp.sum(bp3 * bp3, axis=1, keepdims=True))
    bg_n = jnp.sqrt(jnp.sum(bg3 * bg3, axis=1, keepdims=True))
    bpn = bp3 / (bp_n + 1e-8)
    bgn = bg3 / (bg_n + 1e-8)
    cos = -jnp.sum(bpn * bgn, axis=1)
    acc_ref[4] += jnp.sum(cos * mask)

    logit = _dot(feat, ws_ref[...]) + bs_ref[0:1, :]
    seg = seg_ref[...]
    valid = seg != -1
    validf = valid.astype(_f32)
    tgt = jnp.where(valid, seg, 0)
    col = jax.lax.broadcasted_iota(jnp.int32, logit.shape, 1)
    active = col < NUM_CLS
    neg = jnp.where(active, logit, -jnp.inf)
    m = jnp.max(neg, axis=1)
    z = jnp.sum(jnp.where(active, jnp.exp(neg - m[:, None]), 0.0), axis=1)
    logz = jnp.log(z) + m
    ll = jnp.sum(jnp.where(col == tgt[:, None], logit, 0.0), axis=1)
    acc_ref[0] += jnp.sum((logz - ll) * validf)
    acc_ref[1] += jnp.sum(validf)

    @pl.when(i == nsteps - 1)
    def _():
        seg_loss = acc_ref[0] / jnp.maximum(acc_ref[1], 1.0)
        denom = acc_ref[3] + 1e-8
        l1 = acc_ref[2] / denom
        cosl = acc_ref[4] / denom
        out_ref[0] = seg_loss + l1 + cosl
        out_ref[1] = seg_loss
        out_ref[2] = l1
        out_ref[3] = cosl


def kernel(coord, segment, instance, instance_center, bbox, offset, feat_in,
           W_bb, b_bb, W1, b1, gamma, beta, W2, b2, Ws, bs):
    x_aug = jnp.concatenate(
        [feat_in, jnp.zeros((N, 2), _f32)], axis=1)
    Wbb_pad = jnp.pad(W_bb.astype(_f32), ((0, 2), (0, 0)))
    bbb_row = jnp.broadcast_to(b_bb.astype(_f32)[None, :], (1, C))
    b1_row = jnp.broadcast_to(b1.astype(_f32)[None, :], (1, C))

    stats = pl.pallas_call(
        _stats_kernel,
        grid=(N // TILE,),
        in_specs=[
            pl.BlockSpec((TILE, 8), lambda i: (i, 0)),
            pl.BlockSpec((8, 256), lambda i: (0, 0)),
            pl.BlockSpec((1, 256), lambda i: (0, 0)),
            pl.BlockSpec((256, 256), lambda i: (0, 0)),
            pl.BlockSpec((1, 256), lambda i: (0, 0)),
        ],
        out_specs=pl.BlockSpec((8, 256), lambda i: (0, 0)),
        out_shape=jax.ShapeDtypeStruct((8, 256), _f32),
    )(x_aug, Wbb_pad, bbb_row, W1, b1_row)

    nf = _f32(N)
    mu = stats[0, :] / nf
    var = stats[1, :] / nf - mu * mu
    sd = jnp.sqrt(var + 1e-3)
    bn = jnp.stack([mu, sd, gamma.astype(_f32), beta.astype(_f32)], axis=0)
    bn = jnp.pad(bn, ((0, 4), (0, 0)))

    W2_pad = jnp.pad(W2.astype(_f32), ((0, 0), (0, 5)))
    b2_row = jnp.pad(b2.astype(_f32), (0, 5))[None, :]
    Ws_pad = jnp.pad(Ws.astype(_f32), ((0, 0), (0, 128 - NUM_CLS)))
    bs_row = jnp.pad(bs.astype(_f32), (0, 128 - NUM_CLS))[None, :]

    out = pl.pallas_call(
        _loss_kernel,
        grid=(N // TILE,),
        in_specs=[
            pl.BlockSpec((TILE, 8), lambda i: (i, 0)),
            pl.BlockSpec((TILE, 3), lambda i: (i, 0)),
            pl.BlockSpec((TILE, 3), lambda i: (i, 0)),
            pl.BlockSpec((TILE,), lambda i: (i,)),
            pl.BlockSpec((TILE,), lambda i: (i,)),
            pl.BlockSpec((8, 256), lambda i: (0, 0)),
            pl.BlockSpec((1, 256), lambda i: (0, 0)),
            pl.BlockSpec((256, 256), lambda i: (0, 0)),
            pl.BlockSpec((1, 256), lambda i: (0, 0)),
            pl.BlockSpec((8, 256), lambda i: (0, 0)),
            pl.BlockSpec((256, 8), lambda i: (0, 0)),
            pl.BlockSpec((1, 8), lambda i: (0, 0)),
            pl.BlockSpec((256, 128), lambda i: (0, 0)),
            pl.BlockSpec((1, 128), lambda i: (0, 0)),
        ],
        out_specs=pl.BlockSpec(memory_space=pltpu.SMEM),
        out_shape=jax.ShapeDtypeStruct((4,), _f32),
        scratch_shapes=[pltpu.SMEM((6,), _f32)],
    )(x_aug, coord, instance_center, segment, instance,
      Wbb_pad, bbb_row, W1, b1_row, bn, W2_pad, b2_row, Ws_pad, bs_row)

    return (out[0], out[1], out[2], out[3])


# speedup vs baseline: 5.2125x; 5.2125x over previous
import jax
import jax.numpy as jnp
from jax.experimental import pallas as pl
from jax.experimental.pallas import tpu as pltpu

N = 262144
C_IN = 6
C = 256
NUM_CLS = 20

TILE_A = 32768
TILE = 4096

_f32 = jnp.float32
_HIGH = jax.lax.Precision.HIGHEST


def _gram_kernel(x_ref, out_ref):
    i = pl.program_id(0)
    x = x_ref[...]
    g = jax.lax.dot_general(x, x, (((1,), (1,)), ((), ())),
                            preferred_element_type=_f32,
                            precision=_HIGH)
    g = jnp.pad(g, ((0, 0), (0, 120)))

    @pl.when(i == 0)
    def _():
        out_ref[...] = jnp.zeros_like(out_ref)

    out_ref[...] += g


def _loss_kernel(x_ref, cg_ref, wbb_ref, bbb_ref, w1_ref, b1_ref, bn_ref,
                 w2_ref, b2_ref, ws_ref, bs_ref, out_ref, acc_ref):
    i = pl.program_id(0)
    nsteps = pl.num_programs(0)

    @pl.when(i == 0)
    def _():
        acc_ref[...] = jnp.zeros_like(acc_ref)

    def dot(a, b):
        return jax.lax.dot_general(a, b, (((1,), (0,)), ((), ())),
                                   preferred_element_type=_f32)

    x = x_ref[...]
    feat = dot(wbb_ref[...], x) + bbb_ref[...]
    h = dot(w1_ref[...], feat) + b1_ref[...]
    mu = bn_ref[:, 0:1]
    sd = bn_ref[:, 1:2]
    gam = bn_ref[:, 2:3]
    bet = bn_ref[:, 3:4]
    hn = jnp.maximum((h - mu) / sd * gam + bet, 0.0)
    bp = dot(w2_ref[...], hn) + b2_ref[...]
    bp3 = bp[0:3, :]

    cg = cg_ref[...]
    bg3 = cg[3:6, :] - cg[0:3, :]
    inst = jax.lax.bitcast_convert_type(cg[7:8, :], jnp.int32)
    mask = (inst != -1).astype(_f32)

    bias_dist = jnp.sum(jnp.abs(bp3 - bg3), axis=0, keepdims=True)
    bp_n = jnp.sqrt(jnp.sum(bp3 * bp3, axis=0, keepdims=True))
    bg_n = jnp.sqrt(jnp.sum(bg3 * bg3, axis=0, keepdims=True))
    bpn = bp3 / (bp_n + 1e-8)
    bgn = bg3 / (bg_n + 1e-8)
    cos = -jnp.sum(bpn * bgn, axis=0, keepdims=True)

    logit = dot(ws_ref[...], feat) + bs_ref[...]
    seg = jax.lax.bitcast_convert_type(cg[6:7, :], jnp.int32)
    valid = seg != -1
    validf = valid.astype(_f32)
    tgt = jnp.where(valid, seg, 0)
    row = jax.lax.broadcasted_iota(jnp.int32, logit.shape, 0)
    active = row < NUM_CLS
    neg = jnp.where(active, logit, -jnp.inf)
    m = jnp.max(neg, axis=0, keepdims=True)
    z = jnp.sum(jnp.where(active, jnp.exp(neg - m), 0.0),
                axis=0, keepdims=True)
    logz = jnp.log(z) + m
    ll = jnp.sum(jnp.where(row == tgt, logit, 0.0), axis=0, keepdims=True)
    nll = (logz - ll) * validf

    acc_ref[0:1, :] += nll
    acc_ref[1:2, :] += validf
    acc_ref[2:3, :] += bias_dist * mask
    acc_ref[3:4, :] += mask
    acc_ref[4:5, :] += cos * mask

    @pl.when(i == nsteps - 1)
    def _():
        seg_loss = jnp.sum(acc_ref[0, :]) / jnp.maximum(
            jnp.sum(acc_ref[1, :]), 1.0)
        denom = jnp.sum(acc_ref[3, :]) + 1e-8
        l1 = jnp.sum(acc_ref[2, :]) / denom
        cosl = jnp.sum(acc_ref[4, :]) / denom
        out_ref[0] = seg_loss + l1 + cosl
        out_ref[1] = seg_loss
        out_ref[2] = l1
        out_ref[3] = cosl


def kernel(coord, segment, instance, instance_center, bbox, offset, feat_in,
           W_bb, b_bb, W1, b1, gamma, beta, W2, b2, Ws, bs):
    ones = jnp.ones((N, 1), _f32)
    zeros = jnp.zeros((N, 1), _f32)
    xT = jnp.concatenate([feat_in, ones, zeros], axis=1).T

    segf = jax.lax.bitcast_convert_type(segment, _f32)
    instf = jax.lax.bitcast_convert_type(instance, _f32)
    cg = jnp.stack([coord[:, 0], coord[:, 1], coord[:, 2],
                    instance_center[:, 0], instance_center[:, 1],
                    instance_center[:, 2], segf, instf], axis=0)

    S = pl.pallas_call(
        _gram_kernel,
        grid=(N // TILE_A,),
        in_specs=[pl.BlockSpec((8, TILE_A), lambda i: (0, i))],
        out_specs=pl.BlockSpec((8, 128), lambda i: (0, 0)),
        out_shape=jax.ShapeDtypeStruct((8, 128), _f32),
    )(xT)[:, :8]

    W_aug = jnp.concatenate(
        [(W_bb @ W1).astype(_f32),
         (b_bb @ W1 + b1).astype(_f32)[None, :],
         jnp.zeros((1, C), _f32)], axis=0)
    nf = _f32(N)
    mu = jnp.matmul(S[6, :], W_aug, precision=_HIGH) / nf
    sw = jnp.matmul(S, W_aug, precision=_HIGH)
    ex2 = jnp.sum(W_aug * sw, axis=0) / nf
    var = ex2 - mu * mu
    sd = jnp.sqrt(var + 1e-3)
    bn = jnp.stack([mu, sd, gamma.astype(_f32), beta.astype(_f32)],
                   axis=1)
    bn = jnp.pad(bn, ((0, 0), (0, 4)))

    WbbT = jnp.pad(W_bb.astype(_f32), ((0, 2), (0, 0))).T
    bbb_col = b_bb.astype(_f32)[:, None]
    W1T = W1.astype(_f32).T
    b1_col = b1.astype(_f32)[:, None]
    W2T = jnp.pad(W2.astype(_f32), ((0, 0), (0, 5))).T
    b2_col = jnp.pad(b2.astype(_f32), (0, 5))[:, None]
    WsT = jnp.pad(Ws.astype(_f32), ((0, 0), (0, 12))).T
    bs_col = jnp.pad(bs.astype(_f32), (0, 12))[:, None]

    out = pl.pallas_call(
        _loss_kernel,
        grid=(N // TILE,),
        in_specs=[
            pl.BlockSpec((8, TILE), lambda i: (0, i)),
            pl.BlockSpec((8, TILE), lambda i: (0, i)),
            pl.BlockSpec((256, 8), lambda i: (0, 0)),
            pl.BlockSpec((256, 1), lambda i: (0, 0)),
            pl.BlockSpec((256, 256), lambda i: (0, 0)),
            pl.BlockSpec((256, 1), lambda i: (0, 0)),
            pl.BlockSpec((256, 8), lambda i: (0, 0)),
            pl.BlockSpec((8, 256), lambda i: (0, 0)),
            pl.BlockSpec((8, 1), lambda i: (0, 0)),
            pl.BlockSpec((32, 256), lambda i: (0, 0)),
            pl.BlockSpec((32, 1), lambda i: (0, 0)),
        ],
        out_specs=pl.BlockSpec(memory_space=pltpu.SMEM),
        out_shape=jax.ShapeDtypeStruct((4,), _f32),
        scratch_shapes=[pltpu.VMEM((8, TILE), _f32)],
    )(xT, cg, WbbT, bbb_col, W1T, b1_col, bn, W2T, b2_col, WsT, bs_col)

    return (out[0], out[1], out[2], out[3])


# speedup vs baseline: 6.0545x; 1.1615x over previous
import jax
import jax.numpy as jnp
from jax.experimental import pallas as pl
from jax.experimental.pallas import tpu as pltpu

N = 262144
C_IN = 6
C = 256
NUM_CLS = 20

TILE_A = 32768
TILE = 4096

_f32 = jnp.float32
_HIGH = jax.lax.Precision.HIGHEST


def _gram_kernel(x_ref, out_ref):
    i = pl.program_id(0)
    x = x_ref[...]
    g = jax.lax.dot_general(x, x, (((1,), (1,)), ((), ())),
                            preferred_element_type=_f32,
                            precision=_HIGH)
    g = jnp.pad(g, ((0, 0), (0, 120)))

    @pl.when(i == 0)
    def _():
        out_ref[...] = jnp.zeros_like(out_ref)

    out_ref[...] += g


def _loss_kernel(x_ref, cg_ref, wbb_ref, w1_ref, bn_ref,
                 w2_ref, b2_ref, ws_ref, bs_ref, out_ref, acc_ref):
    i = pl.program_id(0)
    nsteps = pl.num_programs(0)

    @pl.when(i == 0)
    def _():
        acc_ref[...] = jnp.zeros_like(acc_ref)

    def dot(a, b):
        return jax.lax.dot_general(a, b, (((1,), (0,)), ((), ())),
                                   preferred_element_type=_f32)

    x = x_ref[...]
    feat = dot(wbb_ref[...], x)
    h = dot(w1_ref[...], feat)
    hn = jnp.maximum(h * bn_ref[:, 0:1] + bn_ref[:, 1:2], 0.0)
    bp = dot(w2_ref[...], hn) + b2_ref[...]
    bp3 = bp[0:3, :]

    cg = cg_ref[...]
    bg3 = cg[3:6, :] - cg[0:3, :]
    inst = jax.lax.bitcast_convert_type(cg[7:8, :], jnp.int32)
    mask = (inst != -1).astype(_f32)

    bias_dist = jnp.sum(jnp.abs(bp3 - bg3), axis=0, keepdims=True)
    bp_n = jnp.sqrt(jnp.sum(bp3 * bp3, axis=0, keepdims=True))
    bg_n = jnp.sqrt(jnp.sum(bg3 * bg3, axis=0, keepdims=True))
    bpn = bp3 / (bp_n + 1e-8)
    bgn = bg3 / (bg_n + 1e-8)
    cos = -jnp.sum(bpn * bgn, axis=0, keepdims=True)

    logit = dot(ws_ref[...], feat) + bs_ref[...]
    seg = jax.lax.bitcast_convert_type(cg[6:7, :], jnp.int32)
    valid = seg != -1
    validf = valid.astype(_f32)
    tgt = jnp.where(valid, seg, 0)
    row = jax.lax.broadcasted_iota(jnp.int32, logit.shape, 0)
    active = row < NUM_CLS
    neg = jnp.where(active, logit, -jnp.inf)
    m = jnp.max(neg, axis=0, keepdims=True)
    z = jnp.sum(jnp.where(active, jnp.exp(neg - m), 0.0),
                axis=0, keepdims=True)
    logz = jnp.log(z) + m
    ll = jnp.sum(jnp.where(row == tgt, logit, 0.0), axis=0, keepdims=True)
    nll = (logz - ll) * validf

    acc_ref[0:1, :] += nll
    acc_ref[1:2, :] += validf
    acc_ref[2:3, :] += bias_dist * mask
    acc_ref[3:4, :] += mask
    acc_ref[4:5, :] += cos * mask

    @pl.when(i == nsteps - 1)
    def _():
        seg_loss = jnp.sum(acc_ref[0, :]) / jnp.maximum(
            jnp.sum(acc_ref[1, :]), 1.0)
        denom = jnp.sum(acc_ref[3, :]) + 1e-8
        l1 = jnp.sum(acc_ref[2, :]) / denom
        cosl = jnp.sum(acc_ref[4, :]) / denom
        out_ref[0] = seg_loss + l1 + cosl
        out_ref[1] = seg_loss
        out_ref[2] = l1
        out_ref[3] = cosl


def kernel(coord, segment, instance, instance_center, bbox, offset, feat_in,
           W_bb, b_bb, W1, b1, gamma, beta, W2, b2, Ws, bs):
    ones = jnp.ones((N, 1), _f32)
    zeros = jnp.zeros((N, 1), _f32)
    xT = jnp.concatenate([feat_in, ones, zeros], axis=1).T

    segf = jax.lax.bitcast_convert_type(segment, _f32)
    instf = jax.lax.bitcast_convert_type(instance, _f32)
    cg = jnp.stack([coord[:, 0], coord[:, 1], coord[:, 2],
                    instance_center[:, 0], instance_center[:, 1],
                    instance_center[:, 2], segf, instf], axis=0)

    S = pl.pallas_call(
        _gram_kernel,
        grid=(N // TILE_A,),
        in_specs=[pl.BlockSpec((8, TILE_A), lambda i: (0, i))],
        out_specs=pl.BlockSpec((8, 128), lambda i: (0, 0)),
        out_shape=jax.ShapeDtypeStruct((8, 128), _f32),
    )(xT)[:, :8]

    W_aug = jnp.concatenate(
        [(W_bb @ W1).astype(_f32),
         (b_bb @ W1 + b1).astype(_f32)[None, :],
         jnp.zeros((1, C), _f32)], axis=0)
    nf = _f32(N)
    mu = jnp.matmul(S[6, :], W_aug, precision=_HIGH) / nf
    sw = jnp.matmul(S, W_aug, precision=_HIGH)
    ex2 = jnp.sum(W_aug * sw, axis=0) / nf
    var = ex2 - mu * mu
    sd = jnp.sqrt(var + 1e-3)
    A = gamma.astype(_f32) / sd
    B = (b1.astype(_f32) - mu) * A + beta.astype(_f32)
    bn = jnp.stack([A, B], axis=1)
    bn = jnp.pad(bn, ((0, 0), (0, 6)))

    WbbT = jnp.pad(W_bb.astype(_f32), ((0, 2), (0, 0))).T
    WbbT = WbbT.at[:, 6].add(b_bb.astype(_f32))
    W1T = W1.astype(_f32).T
    W2T = jnp.pad(W2.astype(_f32), ((0, 0), (0, 5))).T
    b2_col = jnp.pad(b2.astype(_f32), (0, 5))[:, None]
    WsT = jnp.pad(Ws.astype(_f32), ((0, 0), (0, 12))).T
    bs_col = jnp.pad(bs.astype(_f32), (0, 12))[:, None]

    out = pl.pallas_call(
        _loss_kernel,
        grid=(N // TILE,),
        in_specs=[
            pl.BlockSpec((8, TILE), lambda i: (0, i)),
            pl.BlockSpec((8, TILE), lambda i: (0, i)),
            pl.BlockSpec((256, 8), lambda i: (0, 0)),
            pl.BlockSpec((256, 256), lambda i: (0, 0)),
            pl.BlockSpec((256, 8), lambda i: (0, 0)),
            pl.BlockSpec((8, 256), lambda i: (0, 0)),
            pl.BlockSpec((8, 1), lambda i: (0, 0)),
            pl.BlockSpec((32, 256), lambda i: (0, 0)),
            pl.BlockSpec((32, 1), lambda i: (0, 0)),
        ],
        out_specs=pl.BlockSpec(memory_space=pltpu.SMEM),
        out_shape=jax.ShapeDtypeStruct((4,), _f32),
        scratch_shapes=[pltpu.VMEM((8, TILE), _f32)],
    )(xT, cg, WbbT, W1T, bn, W2T, b2_col, WsT, bs_col)

    return (out[0], out[1], out[2], out[3])
